# baseline (speedup 1.0000x reference)
import functools

import jax
import jax.numpy as jnp
from jax import lax
from jax.experimental import pallas as pl
from jax.experimental.pallas import tpu as pltpu
from jax.experimental.pallas import tpu_sc as plsc

_N = 10000
_E = 320000
_H = 128
_L = 4

_CHUNK = 128
_NC, _NS = 2, 16
_NW = _NC * _NS
_CPW = 80
_SLAB = 40
_EPW = _CPW * _CHUNK
_EPAD = _NW * _EPW
_NROWCH = 79
_NPAD = _NROWCH * _CHUNK
_RPT = 5
_DEGW = 128

_RB = 1000
_NB = _N // _RB

_mesh = plsc.VectorSubcoreMesh(core_axis_name="c", subcore_axis_name="s")


@functools.partial(
    pl.kernel,
    out_type=jax.ShapeDtypeStruct((_NC, _NPAD, _H), jnp.float32),
    mesh=_mesh,
    scratch_types=[
        pltpu.SemaphoreType.DMA,
        pltpu.SemaphoreType.DMA,
        pltpu.VMEM_SHARED((_NPAD, _H), jnp.float32),
    ],
)
def _sc_agg(h_hbm, src_hbm, dst_hbm, zeros_hbm, out_hbm, sem_a, sem_b, shared):
    cid = lax.axis_index("c")
    sid = lax.axis_index("s")
    wid = cid * _NS + sid

    def _run(src_all, dst_all, rows_a, rows_b):
        for k in range(_RPT):
            r = sid + k * _NS

            @pl.when(r < _NROWCH)
            def _():
                r0 = pl.multiple_of(r * _CHUNK, 8)
                pltpu.sync_copy(zeros_hbm, shared.at[pl.ds(r0, _CHUNK)])

        plsc.subcore_barrier()

        for half in range(_CPW // _SLAB):
            cbase = wid * _CPW + half * _SLAB
            pltpu.sync_copy(src_hbm.at[pl.ds(cbase, _SLAB)], src_all)
            pltpu.sync_copy(dst_hbm.at[pl.ds(cbase, _SLAB)], dst_all)

            pltpu.async_copy(h_hbm.at[src_all.at[0]], rows_a, sem_a)
            pltpu.async_copy(h_hbm.at[src_all.at[1]], rows_b, sem_b)

            def body(j, carry):
                c0 = j * 2
                c1 = c0 + 1
                pltpu.make_async_copy(h_hbm.at[src_all.at[c0]], rows_a, sem_a).wait()
                pltpu.sync_copy(rows_a, shared.at[dst_all.at[c0]], add=True)

                @pl.when(j < _SLAB // 2 - 1)
                def _():
                    pltpu.async_copy(h_hbm.at[src_all.at[c0 + 2]], rows_a, sem_a)

                pltpu.make_async_copy(h_hbm.at[src_all.at[c1]], rows_b, sem_b).wait()
                pltpu.sync_copy(rows_b, shared.at[dst_all.at[c1]], add=True)

                @pl.when(j < _SLAB // 2 - 1)
                def _():
                    pltpu.async_copy(h_hbm.at[src_all.at[c1 + 2]], rows_b, sem_b)

                return carry

            lax.fori_loop(0, _SLAB // 2, body, 0)

        plsc.subcore_barrier()

        for k in range(_RPT):
            r = sid + k * _NS

            @pl.when(r < _NROWCH)
            def _():
                r0 = pl.multiple_of(r * _CHUNK, 8)
                pltpu.sync_copy(shared.at[pl.ds(r0, _CHUNK)],
                                out_hbm.at[cid, pl.ds(r0, _CHUNK)])

    pl.run_scoped(_run,
                  pltpu.VMEM((_SLAB, _CHUNK), jnp.int32),
                  pltpu.VMEM((_SLAB, _CHUNK), jnp.int32),
                  pltpu.VMEM((_CHUNK, _H), jnp.float32),
                  pltpu.VMEM((_CHUNK, _H), jnp.float32))


@functools.partial(
    pl.kernel,
    out_type=jax.ShapeDtypeStruct((_NC, _NPAD, _DEGW), jnp.float32),
    mesh=_mesh,
    scratch_types=[
        pltpu.SemaphoreType.DMA,
        pltpu.VMEM_SHARED((_NPAD, _DEGW), jnp.float32),
    ],
)
def _sc_deg(dst_hbm, ones_hbm, zerosw_hbm, out_hbm, sem, shared):
    cid = lax.axis_index("c")
    sid = lax.axis_index("s")
    wid = cid * _NS + sid

    def _run(dst_all, ones_v):
        cbase = wid * _CPW
        pltpu.sync_copy(dst_hbm.at[pl.ds(cbase, _CPW)], dst_all)
        pltpu.sync_copy(ones_hbm, ones_v)
        for k in range(_RPT):
            r = sid + k * _NS

            @pl.when(r < _NROWCH)
            def _():
                r0 = pl.multiple_of(r * _CHUNK, 8)
                pltpu.sync_copy(zerosw_hbm, shared.at[pl.ds(r0, _CHUNK)])

        plsc.subcore_barrier()

        pltpu.async_copy(ones_v, shared.at[dst_all.at[0]], sem, add=True)
        pltpu.async_copy(ones_v, shared.at[dst_all.at[1]], sem, add=True)

        def body(c, carry):
            pltpu.async_copy(ones_v, shared.at[dst_all.at[c]], sem, add=True)
            pltpu.make_async_copy(ones_v, shared.at[dst_all.at[c - 2]], sem).wait()
            return carry

        lax.fori_loop(2, _CPW, body, 0)
        pltpu.make_async_copy(ones_v, shared.at[dst_all.at[_CPW - 2]], sem).wait()
        pltpu.make_async_copy(ones_v, shared.at[dst_all.at[_CPW - 1]], sem).wait()
        plsc.subcore_barrier()

        for k in range(_RPT):
            r = sid + k * _NS

            @pl.when(r < _NROWCH)
            def _():
                r0 = pl.multiple_of(r * _CHUNK, 8)
                pltpu.sync_copy(shared.at[pl.ds(r0, _CHUNK)],
                                out_hbm.at[cid, pl.ds(r0, _CHUNK)])

    pl.run_scoped(_run,
                  pltpu.VMEM((_CPW, _CHUNK), jnp.int32),
                  pltpu.VMEM((_CHUNK, _DEGW), jnp.float32))


def _mm_t(a, w):
    return lax.dot_general(a, w, (((1,), (1,)), ((), ())),
                           preferred_element_type=jnp.float32)


def _embed_body(x_ref, w_ref, b_ref, o_ref):
    o_ref[...] = jnp.maximum(_mm_t(x_ref[...], w_ref[...]) + b_ref[...], 0.0)


def _layer_body(skip, p_ref, d_ref, h_ref, wl_ref, bl_ref, wr_ref, g_ref, b2_ref, o_ref):
    d = d_ref[...]
    deg = jnp.maximum(d[0, :, 0:1] + d[1, :, 0:1], 1.0)
    p = p_ref[...]
    h = h_ref[...]
    agg = (p[0] + p[1]) / deg
    z = _mm_t(agg, wl_ref[...]) + bl_ref[...] + _mm_t(h, wr_ref[...])
    mu = jnp.mean(z, axis=-1, keepdims=True)
    zc = z - mu
    var = jnp.mean(zc * zc, axis=-1, keepdims=True)
    zn = zc * lax.rsqrt(var + 1e-5) * g_ref[...] + b2_ref[...]
    if skip:
        zn = zn + h
    o_ref[...] = jnp.maximum(zn, 0.0)


def _fuse_body(r0_ref, r1_ref, r2_ref, r3_ref, r4_ref,
               wf1_ref, bf1_ref, wf2_ref, bf2_ref, o_ref):
    w1 = wf1_ref[...]
    z = _mm_t(r0_ref[...], w1[:, 0 * _H:1 * _H])
    z += _mm_t(r1_ref[...], w1[:, 1 * _H:2 * _H])
    z += _mm_t(r2_ref[...], w1[:, 2 * _H:3 * _H])
    z += _mm_t(r3_ref[...], w1[:, 3 * _H:4 * _H])
    z += _mm_t(r4_ref[...], w1[:, 4 * _H:5 * _H])
    hh = jnp.maximum(z + bf1_ref[...], 0.0)
    o_ref[...] = _mm_t(hh, wf2_ref[...]) + bf2_ref[...]


def _row_spec(shape):
    return pl.BlockSpec(shape, lambda i: (i,) + (0,) * (len(shape) - 1))


def _full_spec(shape):
    return pl.BlockSpec(shape, lambda i: (0,) * len(shape))


def _tc_embed(x, W_emb, b_emb):
    return pl.pallas_call(
        _embed_body,
        grid=(_NB,),
        in_specs=[_row_spec((_RB, _H)), _full_spec((_H, _H)), _full_spec((1, _H))],
        out_specs=_row_spec((_RB, _H)),
        out_shape=jax.ShapeDtypeStruct((_N, _H), jnp.float32),
    )(x, W_emb, b_emb.reshape(1, _H))


def _tc_layer(skip, p, degp, h, Wl_i, bl_i, Wr_i, g_i, b_i):
    lead3 = pl.BlockSpec((_NC, _RB, _H), lambda i: (0, i, 0))
    lead3d = pl.BlockSpec((_NC, _RB, _DEGW), lambda i: (0, i, 0))
    return pl.pallas_call(
        functools.partial(_layer_body, skip),
        grid=(_NB,),
        in_specs=[lead3, lead3d, _row_spec((_RB, _H)),
                  _full_spec((_H, _H)), _full_spec((1, _H)),
                  _full_spec((_H, _H)), _full_spec((1, _H)), _full_spec((1, _H))],
        out_specs=_row_spec((_RB, _H)),
        out_shape=jax.ShapeDtypeStruct((_N, _H), jnp.float32),
    )(p, degp, h, Wl_i, bl_i.reshape(1, _H), Wr_i, g_i.reshape(1, _H),
      b_i.reshape(1, _H))


def _tc_fuse(reps, Wf1, bf1, Wf2, bf2):
    return pl.pallas_call(
        _fuse_body,
        grid=(_NB,),
        in_specs=[_row_spec((_RB, _H))] * 5 +
                 [_full_spec((_H, _L * _H + _H)), _full_spec((1, _H)),
                  _full_spec((_H, _H)), _full_spec((1, _H))],
        out_specs=_row_spec((_RB, _H)),
        out_shape=jax.ShapeDtypeStruct((_N, _H), jnp.float32),
    )(*reps, Wf1, bf1.reshape(1, _H), Wf2, bf2.reshape(1, _H))


def kernel(x, edge_index, W_emb, b_emb, Wl, bl, Wr, ln_g, ln_b, Wf1, bf1, Wf2, bf2):
    src = edge_index[0]
    dst = edge_index[1]
    pad = _EPAD - _E
    src_p = jnp.concatenate([src, jnp.zeros((pad,), jnp.int32)])
    src_p = src_p.reshape(_NW * _CPW, _CHUNK)
    dst_p = jnp.concatenate([dst, jnp.full((pad,), _N, jnp.int32)])
    dst_p = dst_p.reshape(_NW * _CPW, _CHUNK)
    zeros_h = jnp.zeros((_CHUNK, _H), jnp.float32)
    zeros_w = jnp.zeros((_CHUNK, _DEGW), jnp.float32)
    ones_w = jnp.ones((_CHUNK, _DEGW), jnp.float32)

    degp = _sc_deg(dst_p, ones_w, zeros_w)
    h = _tc_embed(x, W_emb, b_emb)
    reps = [h]
    for i in range(_L):
        p = _sc_agg(h, src_p, dst_p, zeros_h)
        h = _tc_layer(i > 0, p, degp, h, Wl[i], bl[i], Wr[i], ln_g[i], ln_b[i])
        reps.append(h)
    return _tc_fuse(reps, Wf1, bf1, Wf2, bf2)


# speedup vs baseline: 1.1683x; 1.1683x over previous
import functools

import jax
import jax.numpy as jnp
from jax import lax
from jax.experimental import pallas as pl
from jax.experimental.pallas import tpu as pltpu
from jax.experimental.pallas import tpu_sc as plsc

_N = 10000
_E = 320000
_H = 128
_L = 4

_CHUNK = 128
_NC, _NS = 2, 16
_NW = _NC * _NS
_CPW = 80
_SLAB = 40
_EPW = _CPW * _CHUNK
_EPAD = _NW * _EPW
_NROWCH = 79
_NPAD = _NROWCH * _CHUNK
_RPT = 5
_DEGW = 128

_RB = 1000
_NB = _N // _RB

_mesh = plsc.VectorSubcoreMesh(core_axis_name="c", subcore_axis_name="s")


@functools.partial(
    pl.kernel,
    out_type=jax.ShapeDtypeStruct((_NC, _NPAD, _H), jnp.float32),
    mesh=_mesh,
    scratch_types=[
        pltpu.SemaphoreType.DMA,
        pltpu.SemaphoreType.DMA,
        pltpu.VMEM_SHARED((_NPAD, _H), jnp.float32),
    ],
)
def _sc_agg(h_hbm, src_hbm, dst_hbm, zeros_hbm, out_hbm, sem_a, sem_b, shared):
    cid = lax.axis_index("c")
    sid = lax.axis_index("s")
    wid = cid * _NS + sid

    def _run(src_all, dst_all, rows_a, rows_b):
        for k in range(_RPT):
            r = sid + k * _NS

            @pl.when(r < _NROWCH)
            def _():
                r0 = pl.multiple_of(r * _CHUNK, 8)
                pltpu.sync_copy(zeros_hbm, shared.at[pl.ds(r0, _CHUNK)])

        plsc.subcore_barrier()

        for half in range(_CPW // _SLAB):
            cbase = wid * _CPW + half * _SLAB
            pltpu.sync_copy(src_hbm.at[pl.ds(cbase, _SLAB)], src_all)
            pltpu.sync_copy(dst_hbm.at[pl.ds(cbase, _SLAB)], dst_all)

            pltpu.async_copy(h_hbm.at[src_all.at[0]], rows_a, sem_a)
            pltpu.async_copy(h_hbm.at[src_all.at[1]], rows_b, sem_b)

            def body(j, carry):
                c0 = j * 2
                c1 = c0 + 1
                pltpu.make_async_copy(h_hbm.at[src_all.at[c0]], rows_a, sem_a).wait()
                pltpu.sync_copy(rows_a, shared.at[dst_all.at[c0]], add=True)

                @pl.when(j < _SLAB // 2 - 1)
                def _():
                    pltpu.async_copy(h_hbm.at[src_all.at[c0 + 2]], rows_a, sem_a)

                pltpu.make_async_copy(h_hbm.at[src_all.at[c1]], rows_b, sem_b).wait()
                pltpu.sync_copy(rows_b, shared.at[dst_all.at[c1]], add=True)

                @pl.when(j < _SLAB // 2 - 1)
                def _():
                    pltpu.async_copy(h_hbm.at[src_all.at[c1 + 2]], rows_b, sem_b)

                return carry

            lax.fori_loop(0, _SLAB // 2, body, 0)

        plsc.subcore_barrier()

        for k in range(_RPT):
            r = sid + k * _NS

            @pl.when(r < _NROWCH)
            def _():
                r0 = pl.multiple_of(r * _CHUNK, 8)
                pltpu.sync_copy(shared.at[pl.ds(r0, _CHUNK)],
                                out_hbm.at[cid, pl.ds(r0, _CHUNK)])

    pl.run_scoped(_run,
                  pltpu.VMEM((_SLAB, _CHUNK), jnp.int32),
                  pltpu.VMEM((_SLAB, _CHUNK), jnp.int32),
                  pltpu.VMEM((_CHUNK, _H), jnp.float32),
                  pltpu.VMEM((_CHUNK, _H), jnp.float32))


@functools.partial(
    pl.kernel,
    out_type=jax.ShapeDtypeStruct((_NC, _NPAD, _DEGW), jnp.float32),
    mesh=_mesh,
    scratch_types=[
        pltpu.SemaphoreType.DMA,
        pltpu.VMEM_SHARED((_NPAD, _DEGW), jnp.float32),
    ],
)
def _sc_deg(dst_hbm, ones_hbm, zerosw_hbm, out_hbm, sem, shared):
    cid = lax.axis_index("c")
    sid = lax.axis_index("s")
    wid = cid * _NS + sid

    def _run(dst_all, ones_v):
        cbase = wid * _CPW
        pltpu.sync_copy(dst_hbm.at[pl.ds(cbase, _CPW)], dst_all)
        pltpu.sync_copy(ones_hbm, ones_v)
        for k in range(_RPT):
            r = sid + k * _NS

            @pl.when(r < _NROWCH)
            def _():
                r0 = pl.multiple_of(r * _CHUNK, 8)
                pltpu.sync_copy(zerosw_hbm, shared.at[pl.ds(r0, _CHUNK)])

        plsc.subcore_barrier()

        pltpu.async_copy(ones_v, shared.at[dst_all.at[0]], sem, add=True)
        pltpu.async_copy(ones_v, shared.at[dst_all.at[1]], sem, add=True)

        def body(c, carry):
            pltpu.async_copy(ones_v, shared.at[dst_all.at[c]], sem, add=True)
            pltpu.make_async_copy(ones_v, shared.at[dst_all.at[c - 2]], sem).wait()
            return carry

        lax.fori_loop(2, _CPW, body, 0)
        pltpu.make_async_copy(ones_v, shared.at[dst_all.at[_CPW - 2]], sem).wait()
        pltpu.make_async_copy(ones_v, shared.at[dst_all.at[_CPW - 1]], sem).wait()
        plsc.subcore_barrier()

        for k in range(_RPT):
            r = sid + k * _NS

            @pl.when(r < _NROWCH)
            def _():
                r0 = pl.multiple_of(r * _CHUNK, 8)
                pltpu.sync_copy(shared.at[pl.ds(r0, _CHUNK)],
                                out_hbm.at[cid, pl.ds(r0, _CHUNK)])

    pl.run_scoped(_run,
                  pltpu.VMEM((_CPW, _CHUNK), jnp.int32),
                  pltpu.VMEM((_CHUNK, _DEGW), jnp.float32))


def _mm_t(a, w):
    return lax.dot_general(a, w, (((1,), (1,)), ((), ())),
                           preferred_element_type=jnp.float32)


def _embed_body(x_ref, w_ref, b_ref, o_ref):
    o_ref[...] = jnp.maximum(_mm_t(x_ref[...], w_ref[...]) + b_ref[...], 0.0)


def _layer_body(skip, p_ref, d_ref, h_ref, wl_ref, bl_ref, wr_ref, g_ref, b2_ref, o_ref):
    d = d_ref[...]
    deg = jnp.maximum(d[0, :, 0:1] + d[1, :, 0:1], 1.0)
    p = p_ref[...]
    h = h_ref[...]
    agg = (p[0] + p[1]) / deg
    z = _mm_t(agg, wl_ref[...]) + bl_ref[...] + _mm_t(h, wr_ref[...])
    mu = jnp.mean(z, axis=-1, keepdims=True)
    zc = z - mu
    var = jnp.mean(zc * zc, axis=-1, keepdims=True)
    zn = zc * lax.rsqrt(var + 1e-5) * g_ref[...] + b2_ref[...]
    if skip:
        zn = zn + h
    o_ref[...] = jnp.maximum(zn, 0.0)


def _fuse_body(r0_ref, r1_ref, r2_ref, r3_ref, r4_ref,
               wf1_ref, bf1_ref, wf2_ref, bf2_ref, o_ref):
    w1 = wf1_ref[...]
    z = _mm_t(r0_ref[...], w1[:, 0 * _H:1 * _H])
    z += _mm_t(r1_ref[...], w1[:, 1 * _H:2 * _H])
    z += _mm_t(r2_ref[...], w1[:, 2 * _H:3 * _H])
    z += _mm_t(r3_ref[...], w1[:, 3 * _H:4 * _H])
    z += _mm_t(r4_ref[...], w1[:, 4 * _H:5 * _H])
    hh = jnp.maximum(z + bf1_ref[...], 0.0)
    o_ref[...] = _mm_t(hh, wf2_ref[...]) + bf2_ref[...]


def _row_spec(shape):
    return pl.BlockSpec(shape, lambda i: (i,) + (0,) * (len(shape) - 1))


def _full_spec(shape):
    return pl.BlockSpec(shape, lambda i: (0,) * len(shape))


def _tc_embed(x, W_emb, b_emb):
    return pl.pallas_call(
        _embed_body,
        grid=(_NB,),
        in_specs=[_row_spec((_RB, _H)), _full_spec((_H, _H)), _full_spec((1, _H))],
        out_specs=_row_spec((_RB, _H)),
        out_shape=jax.ShapeDtypeStruct((_N, _H), jnp.float32),
    )(x, W_emb, b_emb.reshape(1, _H))


def _tc_layer(skip, p, degp, h, Wl_i, bl_i, Wr_i, g_i, b_i):
    lead3 = pl.BlockSpec((_NC, _RB, _H), lambda i: (0, i, 0))
    lead3d = pl.BlockSpec((_NC, _RB, _DEGW), lambda i: (0, i, 0))
    return pl.pallas_call(
        functools.partial(_layer_body, skip),
        grid=(_NB,),
        in_specs=[lead3, lead3d, _row_spec((_RB, _H)),
                  _full_spec((_H, _H)), _full_spec((1, _H)),
                  _full_spec((_H, _H)), _full_spec((1, _H)), _full_spec((1, _H))],
        out_specs=_row_spec((_RB, _H)),
        out_shape=jax.ShapeDtypeStruct((_N, _H), jnp.float32),
    )(p, degp, h, Wl_i, bl_i.reshape(1, _H), Wr_i, g_i.reshape(1, _H),
      b_i.reshape(1, _H))


def _tc_fuse(reps, Wf1, bf1, Wf2, bf2):
    return pl.pallas_call(
        _fuse_body,
        grid=(_NB,),
        in_specs=[_row_spec((_RB, _H))] * 5 +
                 [_full_spec((_H, _L * _H + _H)), _full_spec((1, _H)),
                  _full_spec((_H, _H)), _full_spec((1, _H))],
        out_specs=_row_spec((_RB, _H)),
        out_shape=jax.ShapeDtypeStruct((_N, _H), jnp.float32),
    )(*reps, Wf1, bf1.reshape(1, _H), Wf2, bf2.reshape(1, _H))


def kernel(x, edge_index, W_emb, b_emb, Wl, bl, Wr, ln_g, ln_b, Wf1, bf1, Wf2, bf2):
    src = edge_index[0]
    dst = edge_index[1]
    ppw = _EPW - _E // _NW
    pad_src = jnp.zeros((_NW, ppw), jnp.int32)
    pad_dst = jnp.broadcast_to(_N + (jnp.arange(ppw, dtype=jnp.int32) % (_NPAD - _N)),
                               (_NW, ppw))
    src_p = jnp.concatenate([src.reshape(_NW, _E // _NW), pad_src], axis=1)
    src_p = src_p.reshape(_NW * _CPW, _CHUNK)
    dst_p = jnp.concatenate([dst.reshape(_NW, _E // _NW), pad_dst], axis=1)
    dst_p = dst_p.reshape(_NW * _CPW, _CHUNK)
    zeros_h = jnp.zeros((_CHUNK, _H), jnp.float32)
    zeros_w = jnp.zeros((_CHUNK, _DEGW), jnp.float32)
    ones_w = jnp.ones((_CHUNK, _DEGW), jnp.float32)

    degp = _sc_deg(dst_p, ones_w, zeros_w)
    h = _tc_embed(x, W_emb, b_emb)
    reps = [h]
    for i in range(_L):
        p = _sc_agg(h, src_p, dst_p, zeros_h)
        h = _tc_layer(i > 0, p, degp, h, Wl[i], bl[i], Wr[i], ln_g[i], ln_b[i])
        reps.append(h)
    return _tc_fuse(reps, Wf1, bf1, Wf2, bf2)


# speedup vs baseline: 3.0844x; 2.6401x over previous
import functools

import jax
import jax.numpy as jnp
from jax import lax
from jax.experimental import pallas as pl
from jax.experimental.pallas import tpu as pltpu
from jax.experimental.pallas import tpu_sc as plsc

_N = 10000
_E = 320000
_H = 128
_L = 4

_CHUNK = 128
_NC, _NS = 2, 16
_NW = _NC * _NS
_CPW = 80
_SLAB = 40
_EPW = _CPW * _CHUNK
_EPAD = _NW * _EPW
_NROWCH = 79
_NPAD = _NROWCH * _CHUNK
_RPT = 5
_DEGW = 128

_RB = 1000
_NB = _N // _RB

_mesh = plsc.VectorSubcoreMesh(core_axis_name="c", subcore_axis_name="s")


@functools.partial(
    pl.kernel,
    out_type=jax.ShapeDtypeStruct((_NC, _NPAD, _H), jnp.float32),
    mesh=_mesh,
    scratch_types=[
        pltpu.SemaphoreType.DMA,
        pltpu.SemaphoreType.DMA,
        pltpu.VMEM_SHARED((_NPAD, _H), jnp.float32),
    ],
)
def _sc_agg(h_hbm, src_hbm, dst_hbm, zeros_hbm, out_hbm, sem_a, sem_b, shared):
    cid = lax.axis_index("c")
    sid = lax.axis_index("s")
    wid = cid * _NS + sid

    def _run(src_all, dst_all, rows_a, rows_b):
        for k in range(_RPT):
            r = sid + k * _NS

            @pl.when(r < _NROWCH)
            def _():
                r0 = pl.multiple_of(r * _CHUNK, 8)
                pltpu.sync_copy(zeros_hbm, shared.at[pl.ds(r0, _CHUNK)])

        plsc.subcore_barrier()

        for half in range(_CPW // _SLAB):
            cbase = wid * _CPW + half * _SLAB
            pltpu.sync_copy(src_hbm.at[pl.ds(cbase, _SLAB)], src_all)
            pltpu.sync_copy(dst_hbm.at[pl.ds(cbase, _SLAB)], dst_all)

            pltpu.async_copy(h_hbm.at[src_all.at[0]], rows_a, sem_a)
            pltpu.async_copy(h_hbm.at[src_all.at[1]], rows_b, sem_b)

            def body(j, carry):
                c0 = j * 2
                c1 = c0 + 1
                pltpu.make_async_copy(h_hbm.at[src_all.at[c0]], rows_a, sem_a).wait()
                pltpu.sync_copy(rows_a, shared.at[dst_all.at[c0]], add=True)

                @pl.when(j < _SLAB // 2 - 1)
                def _():
                    pltpu.async_copy(h_hbm.at[src_all.at[c0 + 2]], rows_a, sem_a)

                pltpu.make_async_copy(h_hbm.at[src_all.at[c1]], rows_b, sem_b).wait()
                pltpu.sync_copy(rows_b, shared.at[dst_all.at[c1]], add=True)

                @pl.when(j < _SLAB // 2 - 1)
                def _():
                    pltpu.async_copy(h_hbm.at[src_all.at[c1 + 2]], rows_b, sem_b)

                return carry

            lax.fori_loop(0, _SLAB // 2, body, 0)

        plsc.subcore_barrier()

        for k in range(_RPT):
            r = sid + k * _NS

            @pl.when(r < _NROWCH)
            def _():
                r0 = pl.multiple_of(r * _CHUNK, 8)
                pltpu.sync_copy(shared.at[pl.ds(r0, _CHUNK)],
                                out_hbm.at[cid, pl.ds(r0, _CHUNK)])

    pl.run_scoped(_run,
                  pltpu.VMEM((_SLAB, _CHUNK), jnp.int32),
                  pltpu.VMEM((_SLAB, _CHUNK), jnp.int32),
                  pltpu.VMEM((_CHUNK, _H), jnp.float32),
                  pltpu.VMEM((_CHUNK, _H), jnp.float32))


@functools.partial(
    pl.kernel,
    out_type=jax.ShapeDtypeStruct((_NC, _NPAD, _DEGW), jnp.float32),
    mesh=_mesh,
    scratch_types=[
        pltpu.SemaphoreType.DMA,
        pltpu.VMEM_SHARED((_NPAD, _DEGW), jnp.float32),
    ],
)
def _sc_deg(dst_hbm, ones_hbm, zerosw_hbm, out_hbm, sem, shared):
    cid = lax.axis_index("c")
    sid = lax.axis_index("s")
    wid = cid * _NS + sid

    def _run(dst_all, ones_v):
        cbase = wid * _CPW
        pltpu.sync_copy(dst_hbm.at[pl.ds(cbase, _CPW)], dst_all)
        pltpu.sync_copy(ones_hbm, ones_v)
        for k in range(_RPT):
            r = sid + k * _NS

            @pl.when(r < _NROWCH)
            def _():
                r0 = pl.multiple_of(r * _CHUNK, 8)
                pltpu.sync_copy(zerosw_hbm, shared.at[pl.ds(r0, _CHUNK)])

        plsc.subcore_barrier()

        pltpu.async_copy(ones_v, shared.at[dst_all.at[0]], sem, add=True)
        pltpu.async_copy(ones_v, shared.at[dst_all.at[1]], sem, add=True)

        def body(c, carry):
            pltpu.async_copy(ones_v, shared.at[dst_all.at[c]], sem, add=True)
            pltpu.make_async_copy(ones_v, shared.at[dst_all.at[c - 2]], sem).wait()
            return carry

        lax.fori_loop(2, _CPW, body, 0)
        pltpu.make_async_copy(ones_v, shared.at[dst_all.at[_CPW - 2]], sem).wait()
        pltpu.make_async_copy(ones_v, shared.at[dst_all.at[_CPW - 1]], sem).wait()
        plsc.subcore_barrier()

        for k in range(_RPT):
            r = sid + k * _NS

            @pl.when(r < _NROWCH)
            def _():
                r0 = pl.multiple_of(r * _CHUNK, 8)
                pltpu.sync_copy(shared.at[pl.ds(r0, _CHUNK)],
                                out_hbm.at[cid, pl.ds(r0, _CHUNK)])

    pl.run_scoped(_run,
                  pltpu.VMEM((_CPW, _CHUNK), jnp.int32),
                  pltpu.VMEM((_CHUNK, _DEGW), jnp.float32))


def _mm_t(a, w):
    return lax.dot_general(a, w, (((1,), (1,)), ((), ())),
                           preferred_element_type=jnp.float32)


def _embed_body(x_ref, w_ref, b_ref, o_ref):
    o_ref[...] = jnp.maximum(_mm_t(x_ref[...], w_ref[...]) + b_ref[...], 0.0)


def _layer_body(skip, p_ref, d_ref, h_ref, wl_ref, bl_ref, wr_ref, g_ref, b2_ref, o_ref):
    d = d_ref[...]
    deg = jnp.maximum(d[0, :, 0:1] + d[1, :, 0:1], 1.0)
    p = p_ref[...]
    h = h_ref[...]
    agg = (p[0] + p[1]) / deg
    z = _mm_t(agg, wl_ref[...]) + bl_ref[...] + _mm_t(h, wr_ref[...])
    mu = jnp.mean(z, axis=-1, keepdims=True)
    zc = z - mu
    var = jnp.mean(zc * zc, axis=-1, keepdims=True)
    zn = zc * lax.rsqrt(var + 1e-5) * g_ref[...] + b2_ref[...]
    if skip:
        zn = zn + h
    o_ref[...] = jnp.maximum(zn, 0.0)


def _fuse_body(r0_ref, r1_ref, r2_ref, r3_ref, r4_ref,
               wf1_ref, bf1_ref, wf2_ref, bf2_ref, o_ref):
    w1 = wf1_ref[...]
    z = _mm_t(r0_ref[...], w1[:, 0 * _H:1 * _H])
    z += _mm_t(r1_ref[...], w1[:, 1 * _H:2 * _H])
    z += _mm_t(r2_ref[...], w1[:, 2 * _H:3 * _H])
    z += _mm_t(r3_ref[...], w1[:, 3 * _H:4 * _H])
    z += _mm_t(r4_ref[...], w1[:, 4 * _H:5 * _H])
    hh = jnp.maximum(z + bf1_ref[...], 0.0)
    o_ref[...] = _mm_t(hh, wf2_ref[...]) + bf2_ref[...]


def _row_spec(shape):
    return pl.BlockSpec(shape, lambda i: (i,) + (0,) * (len(shape) - 1))


def _full_spec(shape):
    return pl.BlockSpec(shape, lambda i: (0,) * len(shape))


def _tc_embed(x, W_emb, b_emb):
    return pl.pallas_call(
        _embed_body,
        grid=(_NB,),
        in_specs=[_row_spec((_RB, _H)), _full_spec((_H, _H)), _full_spec((1, _H))],
        out_specs=_row_spec((_RB, _H)),
        out_shape=jax.ShapeDtypeStruct((_N, _H), jnp.float32),
    )(x, W_emb, b_emb.reshape(1, _H))


def _tc_layer(skip, p, degp, h, Wl_i, bl_i, Wr_i, g_i, b_i):
    lead3 = pl.BlockSpec((_NC, _RB, _H), lambda i: (0, i, 0))
    lead3d = pl.BlockSpec((_NC, _RB, _DEGW), lambda i: (0, i, 0))
    return pl.pallas_call(
        functools.partial(_layer_body, skip),
        grid=(_NB,),
        in_specs=[lead3, lead3d, _row_spec((_RB, _H)),
                  _full_spec((_H, _H)), _full_spec((1, _H)),
                  _full_spec((_H, _H)), _full_spec((1, _H)), _full_spec((1, _H))],
        out_specs=_row_spec((_RB, _H)),
        out_shape=jax.ShapeDtypeStruct((_N, _H), jnp.float32),
    )(p, degp, h, Wl_i, bl_i.reshape(1, _H), Wr_i, g_i.reshape(1, _H),
      b_i.reshape(1, _H))


def _tc_fuse(reps, Wf1, bf1, Wf2, bf2):
    return pl.pallas_call(
        _fuse_body,
        grid=(_NB,),
        in_specs=[_row_spec((_RB, _H))] * 5 +
                 [_full_spec((_H, _L * _H + _H)), _full_spec((1, _H)),
                  _full_spec((_H, _H)), _full_spec((1, _H))],
        out_specs=_row_spec((_RB, _H)),
        out_shape=jax.ShapeDtypeStruct((_N, _H), jnp.float32),
    )(*reps, Wf1, bf1.reshape(1, _H), Wf2, bf2.reshape(1, _H))


def kernel(x, edge_index, W_emb, b_emb, Wl, bl, Wr, ln_g, ln_b, Wf1, bf1, Wf2, bf2):
    src = edge_index[0]
    dst = edge_index[1]
    ppw = _EPW - _E // _NW
    pad_src = jnp.broadcast_to(jnp.arange(ppw, dtype=jnp.int32) % _N, (_NW, ppw))
    pad_dst = jnp.broadcast_to(_N + (jnp.arange(ppw, dtype=jnp.int32) % (_NPAD - _N)),
                               (_NW, ppw))
    src_p = jnp.concatenate([src.reshape(_NW, _E // _NW), pad_src], axis=1)
    src_p = src_p.reshape(_NW * _CPW, _CHUNK)
    dst_p = jnp.concatenate([dst.reshape(_NW, _E // _NW), pad_dst], axis=1)
    dst_p = dst_p.reshape(_NW * _CPW, _CHUNK)
    zeros_h = jnp.zeros((_CHUNK, _H), jnp.float32)
    zeros_w = jnp.zeros((_CHUNK, _DEGW), jnp.float32)
    ones_w = jnp.ones((_CHUNK, _DEGW), jnp.float32)

    degp = _sc_deg(dst_p, ones_w, zeros_w)
    h = _tc_embed(x, W_emb, b_emb)
    reps = [h]
    for i in range(_L):
        p = _sc_agg(h, src_p, dst_p, zeros_h)
        h = _tc_layer(i > 0, p, degp, h, Wl[i], bl[i], Wr[i], ln_g[i], ln_b[i])
        reps.append(h)
    return _tc_fuse(reps, Wf1, bf1, Wf2, bf2)
